# Initial kernel scaffold; baseline (speedup 1.0000x reference)
import jax
import jax.numpy as jnp
from jax.experimental import pallas as pl


def kernel(tenInput, tenFlow, tenMetric):
    raise NotImplementedError("write your pallas kernel here")


# speedup vs baseline: 1.1156x; 1.1156x over previous
import jax
import jax.numpy as jnp
from jax import lax
from jax.experimental import pallas as pl
from jax.experimental.pallas import tpu as pltpu
from jax.experimental.pallas import tpu_sc as plsc

B = 2
C = 96
H = 384
W = 384
HW = H * W
NT = 16
SRC = HW // NT
CHUNK = 128
NCHUNK = SRC // CHUNK
LANES = 16
VPC = CHUNK // LANES
EPS = 1e-7


def _sc_body(inp, flow, met, out, idx4, wm4, inbuf, stage, dinv, acc, den):
    b = lax.axis_index("c")
    t = lax.axis_index("s")
    base = t * SRC

    zf = jnp.full((LANES,), 0.0, dtype=jnp.float32)

    def _fill_zero(buf):
        def _z(i, _):
            buf[pl.ds(i * LANES, LANES)] = zf
            return 0
        lax.fori_loop(0, SRC // LANES, _z, 0)

    _fill_zero(inbuf)
    pltpu.sync_copy(inbuf, acc.at[pl.ds(base, SRC)])
    pltpu.sync_copy(inbuf, den.at[pl.ds(base, SRC)])

    pltpu.sync_copy(flow.at[b, 0, pl.ds(base, SRC)], inbuf)
    pltpu.sync_copy(flow.at[b, 1, pl.ds(base, SRC)], stage)
    pltpu.sync_copy(met.at[b, pl.ds(base, SRC)], dinv)

    iota = lax.iota(jnp.int32, LANES)

    def _pre(j, _):
        for q in range(VPC):
            i = j * VPC + q
            p0 = base + i * LANES
            sl = pl.ds(q * LANES, LANES)
            vsl = pl.ds(i * LANES, LANES)
            fx = ((p0 % W) + iota).astype(jnp.float32) + inbuf[vsl]
            fy = (p0 // W).astype(jnp.float32) + stage[vsl]
            x0 = fx.astype(jnp.int32)
            x0f = x0.astype(jnp.float32)
            bx = x0f > fx
            x0 = jnp.where(bx, x0 - 1, x0)
            x0f = jnp.where(bx, x0f - 1.0, x0f)
            y0 = fy.astype(jnp.int32)
            y0f = y0.astype(jnp.float32)
            by = y0f > fy
            y0 = jnp.where(by, y0 - 1, y0)
            y0f = jnp.where(by, y0f - 1.0, y0f)
            ax = fx - x0f
            ay = fy - y0f
            nx = 1.0 - ax
            ny = 1.0 - ay
            m = jnp.exp(dinv[vsl])
            x1 = x0 + 1
            y1 = y0 + 1
            vx0 = (x0 >= 0) & (x0 < W)
            vx1 = (x1 >= 0) & (x1 < W)
            vy0 = (y0 >= 0) & (y0 < H)
            vy1 = (y1 >= 0) & (y1 < H)
            cx0 = jnp.clip(x0, 0, W - 1)
            cx1 = jnp.clip(x1, 0, W - 1)
            cy0 = jnp.clip(y0, 0, H - 1) * W
            cy1 = jnp.clip(y1, 0, H - 1) * W
            idx4[0, j, sl] = cy0 + cx0
            idx4[1, j, sl] = cy0 + cx1
            idx4[2, j, sl] = cy1 + cx0
            idx4[3, j, sl] = cy1 + cx1
            wm4[0, j, sl] = jnp.where(vx0 & vy0, nx * ny * m, zf)
            wm4[1, j, sl] = jnp.where(vx1 & vy0, ax * ny * m, zf)
            wm4[2, j, sl] = jnp.where(vx0 & vy1, nx * ay * m, zf)
            wm4[3, j, sl] = jnp.where(vx1 & vy1, ax * ay * m, zf)
        return 0

    lax.fori_loop(0, NCHUNK, _pre, 0)

    plsc.subcore_barrier()

    for k in range(4):
        def _dscat(j, _, k=k):
            pltpu.sync_copy(wm4.at[k, j], den.at[idx4.at[k, j]], add=True)
            return 0

        lax.fori_loop(0, NCHUNK, _dscat, 0)

    plsc.subcore_barrier()

    pltpu.sync_copy(den.at[pl.ds(base, SRC)], dinv)
    one = jnp.full((LANES,), 1.0, jnp.float32)
    epsv = jnp.full((LANES,), EPS, jnp.float32)

    def _inv(i, _):
        sl = pl.ds(i * LANES, LANES)
        dinv[sl] = one / (dinv[sl] + epsv)
        return 0

    lax.fori_loop(0, SRC // LANES, _inv, 0)

    def _chan(c, _):
        pltpu.sync_copy(inp.at[b, c, pl.ds(base, SRC)], inbuf)
        for k in range(4):
            def _scat(j, _, k=k):
                for q in range(VPC):
                    o = j * CHUNK + q * LANES
                    stage[pl.ds(o, LANES)] = (
                        inbuf[pl.ds(o, LANES)] * wm4[k, j, pl.ds(q * LANES, LANES)])
                pltpu.sync_copy(stage.at[pl.ds(j * CHUNK, CHUNK)],
                                acc.at[idx4.at[k, j]], add=True)
                return 0

            lax.fori_loop(0, NCHUNK, _scat, 0)
        plsc.subcore_barrier()
        pltpu.sync_copy(acc.at[pl.ds(base, SRC)], inbuf)

        def _nrm(i, _):
            sl = pl.ds(i * LANES, LANES)
            inbuf[sl] = inbuf[sl] * dinv[sl]
            return 0

        lax.fori_loop(0, SRC // LANES, _nrm, 0)
        pltpu.sync_copy(inbuf, out.at[b, c, pl.ds(base, SRC)])
        _fill_zero(inbuf)
        pltpu.sync_copy(inbuf, acc.at[pl.ds(base, SRC)])
        plsc.subcore_barrier()
        return 0

    lax.fori_loop(0, C, _chan, 0)


def _softsplat_sc(inp, flow, met):
    mesh = plsc.VectorSubcoreMesh(
        core_axis_name="c", subcore_axis_name="s",
        num_cores=2, num_subcores=16)
    f = pl.kernel(
        _sc_body,
        out_type=jax.ShapeDtypeStruct((B, C, HW), jnp.float32),
        mesh=mesh,
        scratch_types=[
            pltpu.VMEM((4, NCHUNK, CHUNK), jnp.int32),
            pltpu.VMEM((4, NCHUNK, CHUNK), jnp.float32),
            pltpu.VMEM((SRC,), jnp.float32),
            pltpu.VMEM((SRC,), jnp.float32),
            pltpu.VMEM((SRC,), jnp.float32),
            pltpu.VMEM_SHARED((HW,), jnp.float32),
            pltpu.VMEM_SHARED((HW,), jnp.float32),
        ],
    )
    return f(inp, flow, met)


def kernel(tenInput, tenFlow, tenMetric):
    inp = tenInput.reshape(B, C, HW)
    flow = tenFlow.reshape(B, 2, HW)
    met = tenMetric.reshape(B, HW)
    out = _softsplat_sc(inp, flow, met)
    return out.reshape(B, C, H, W)


# speedup vs baseline: 1.8095x; 1.6220x over previous
import jax
import jax.numpy as jnp
from jax import lax
from jax.experimental import pallas as pl
from jax.experimental.pallas import tpu as pltpu
from jax.experimental.pallas import tpu_sc as plsc

B = 2
C = 96
H = 384
W = 384
HW = H * W
NT = 16
SRC = HW // NT
CHUNK = 128
NCHUNK = SRC // CHUNK
LANES = 16
VPC = CHUNK // LANES
EPS = 1e-7


def _sc_body(inp, flow, met, out, idx4, wm4, inbuf, stage, dinv, acc, den,
             sem_a):
    b = lax.axis_index("c")
    t = lax.axis_index("s")
    base = t * SRC

    zf = jnp.full((LANES,), 0.0, dtype=jnp.float32)

    def _fill_zero(buf):
        def _z(i, _):
            buf[pl.ds(i * LANES, LANES)] = zf
            return 0
        lax.fori_loop(0, SRC // LANES, _z, 0)

    _fill_zero(inbuf)
    pltpu.sync_copy(inbuf, acc.at[pl.ds(base, SRC)])
    pltpu.sync_copy(inbuf, den.at[pl.ds(base, SRC)])

    pltpu.sync_copy(flow.at[b, 0, pl.ds(base, SRC)], inbuf)
    pltpu.sync_copy(flow.at[b, 1, pl.ds(base, SRC)], stage)
    pltpu.sync_copy(met.at[b, pl.ds(base, SRC)], dinv)

    iota = lax.iota(jnp.int32, LANES)

    def _pre(j, _):
        for q in range(VPC):
            i = j * VPC + q
            p0 = base + i * LANES
            sl = pl.ds(q * LANES, LANES)
            vsl = pl.ds(i * LANES, LANES)
            fx = ((p0 % W) + iota).astype(jnp.float32) + inbuf[vsl]
            fy = (p0 // W).astype(jnp.float32) + stage[vsl]
            x0 = fx.astype(jnp.int32)
            x0f = x0.astype(jnp.float32)
            bx = x0f > fx
            x0 = jnp.where(bx, x0 - 1, x0)
            x0f = jnp.where(bx, x0f - 1.0, x0f)
            y0 = fy.astype(jnp.int32)
            y0f = y0.astype(jnp.float32)
            by = y0f > fy
            y0 = jnp.where(by, y0 - 1, y0)
            y0f = jnp.where(by, y0f - 1.0, y0f)
            ax = fx - x0f
            ay = fy - y0f
            nx = 1.0 - ax
            ny = 1.0 - ay
            m = jnp.exp(dinv[vsl])
            x1 = x0 + 1
            y1 = y0 + 1
            vx0 = (x0 >= 0) & (x0 < W)
            vx1 = (x1 >= 0) & (x1 < W)
            vy0 = (y0 >= 0) & (y0 < H)
            vy1 = (y1 >= 0) & (y1 < H)
            cx0 = jnp.clip(x0, 0, W - 1)
            cx1 = jnp.clip(x1, 0, W - 1)
            cy0 = jnp.clip(y0, 0, H - 1) * W
            cy1 = jnp.clip(y1, 0, H - 1) * W
            idx4[0, j, sl] = cy0 + cx0
            idx4[1, j, sl] = cy0 + cx1
            idx4[2, j, sl] = cy1 + cx0
            idx4[3, j, sl] = cy1 + cx1
            wm4[0, j, sl] = jnp.where(vx0 & vy0, nx * ny * m, zf)
            wm4[1, j, sl] = jnp.where(vx1 & vy0, ax * ny * m, zf)
            wm4[2, j, sl] = jnp.where(vx0 & vy1, nx * ay * m, zf)
            wm4[3, j, sl] = jnp.where(vx1 & vy1, ax * ay * m, zf)
        return 0

    lax.fori_loop(0, NCHUNK, _pre, 0)

    plsc.subcore_barrier()

    for k in range(4):
        def _dscat(jj, _, k=k):
            j = jj * 2
            d0 = pltpu.async_copy(wm4.at[k, j], den.at[idx4.at[k, j]],
                                  sem_a, add=True)
            d1 = pltpu.async_copy(wm4.at[k, j + 1], den.at[idx4.at[k, j + 1]],
                                  sem_a, add=True)
            d0.wait()
            d1.wait()
            return 0

        lax.fori_loop(0, NCHUNK // 2, _dscat, 0)

    plsc.subcore_barrier()

    pltpu.sync_copy(den.at[pl.ds(base, SRC)], dinv)
    one = jnp.full((LANES,), 1.0, jnp.float32)
    epsv = jnp.full((LANES,), EPS, jnp.float32)

    def _inv(i, _):
        sl = pl.ds(i * LANES, LANES)
        dinv[sl] = one / (dinv[sl] + epsv)
        return 0

    lax.fori_loop(0, SRC // LANES, _inv, 0)

    def _chan(c, _):
        pltpu.sync_copy(inp.at[b, c, pl.ds(base, SRC)], inbuf)
        for k in range(4):
            def _scat(jj, _, k=k):
                j = jj * 4
                ds = []
                for jo in range(4):
                    for q in range(VPC):
                        o = (j + jo) * CHUNK + q * LANES
                        stage[pl.ds(o, LANES)] = (
                            inbuf[pl.ds(o, LANES)]
                            * wm4[k, j + jo, pl.ds(q * LANES, LANES)])
                    ds.append(pltpu.async_copy(
                        stage.at[pl.ds((j + jo) * CHUNK, CHUNK)],
                        acc.at[idx4.at[k, j + jo]], sem_a, add=True))
                for d in ds:
                    d.wait()
                return 0

            lax.fori_loop(0, NCHUNK // 4, _scat, 0)
        plsc.subcore_barrier()
        pltpu.sync_copy(acc.at[pl.ds(base, SRC)], inbuf)

        def _nrm(i, _):
            sl = pl.ds(i * LANES, LANES)
            inbuf[sl] = inbuf[sl] * dinv[sl]
            return 0

        lax.fori_loop(0, SRC // LANES, _nrm, 0)
        pltpu.sync_copy(inbuf, out.at[b, c, pl.ds(base, SRC)])
        _fill_zero(inbuf)
        pltpu.sync_copy(inbuf, acc.at[pl.ds(base, SRC)])
        plsc.subcore_barrier()
        return 0

    lax.fori_loop(0, C, _chan, 0)


def _softsplat_sc(inp, flow, met):
    mesh = plsc.VectorSubcoreMesh(
        core_axis_name="c", subcore_axis_name="s",
        num_cores=2, num_subcores=16)
    f = pl.kernel(
        _sc_body,
        out_type=jax.ShapeDtypeStruct((B, C, HW), jnp.float32),
        mesh=mesh,
        scratch_types=[
            pltpu.VMEM((4, NCHUNK, CHUNK), jnp.int32),
            pltpu.VMEM((4, NCHUNK, CHUNK), jnp.float32),
            pltpu.VMEM((SRC,), jnp.float32),
            pltpu.VMEM((SRC,), jnp.float32),
            pltpu.VMEM((SRC,), jnp.float32),
            pltpu.VMEM_SHARED((HW,), jnp.float32),
            pltpu.VMEM_SHARED((HW,), jnp.float32),
            pltpu.SemaphoreType.DMA,
        ],
    )
    return f(inp, flow, met)


def kernel(tenInput, tenFlow, tenMetric):
    inp = tenInput.reshape(B, C, HW)
    flow = tenFlow.reshape(B, 2, HW)
    met = tenMetric.reshape(B, HW)
    out = _softsplat_sc(inp, flow, met)
    return out.reshape(B, C, H, W)


# speedup vs baseline: 2.0222x; 1.1175x over previous
import jax
import jax.numpy as jnp
from jax import lax
from jax.experimental import pallas as pl
from jax.experimental.pallas import tpu as pltpu
from jax.experimental.pallas import tpu_sc as plsc

B = 2
C = 96
H = 384
W = 384
HW = H * W
NT = 16
SRC = HW // NT
CHUNK = 128
NCHUNK = SRC // CHUNK
LANES = 16
VPC = CHUNK // LANES
EPS = 1e-7


def _sc_body(inp, flow, met, out, idx4, wm4, inbuf, stage, dinv, acc, den,
             sem_a):
    b = lax.axis_index("c")
    t = lax.axis_index("s")
    base = t * SRC

    zf = jnp.full((LANES,), 0.0, dtype=jnp.float32)

    def _fill_zero(buf):
        def _z(i, _):
            buf[pl.ds(i * LANES, LANES)] = zf
            return 0
        lax.fori_loop(0, SRC // LANES, _z, 0)

    _fill_zero(inbuf)
    pltpu.sync_copy(inbuf, acc.at[pl.ds(base, SRC)])
    pltpu.sync_copy(inbuf, den.at[pl.ds(base, SRC)])

    pltpu.sync_copy(flow.at[b, 0, pl.ds(base, SRC)], inbuf)
    pltpu.sync_copy(flow.at[b, 1, pl.ds(base, SRC)], stage)
    pltpu.sync_copy(met.at[b, pl.ds(base, SRC)], dinv)

    iota = lax.iota(jnp.int32, LANES)

    def _pre(j, _):
        for q in range(VPC):
            i = j * VPC + q
            p0 = base + i * LANES
            sl = pl.ds(q * LANES, LANES)
            vsl = pl.ds(i * LANES, LANES)
            fx = ((p0 % W) + iota).astype(jnp.float32) + inbuf[vsl]
            fy = (p0 // W).astype(jnp.float32) + stage[vsl]
            x0 = fx.astype(jnp.int32)
            x0f = x0.astype(jnp.float32)
            bx = x0f > fx
            x0 = jnp.where(bx, x0 - 1, x0)
            x0f = jnp.where(bx, x0f - 1.0, x0f)
            y0 = fy.astype(jnp.int32)
            y0f = y0.astype(jnp.float32)
            by = y0f > fy
            y0 = jnp.where(by, y0 - 1, y0)
            y0f = jnp.where(by, y0f - 1.0, y0f)
            ax = fx - x0f
            ay = fy - y0f
            nx = 1.0 - ax
            ny = 1.0 - ay
            m = jnp.exp(dinv[vsl])
            x1 = x0 + 1
            y1 = y0 + 1
            vx0 = (x0 >= 0) & (x0 < W)
            vx1 = (x1 >= 0) & (x1 < W)
            vy0 = (y0 >= 0) & (y0 < H)
            vy1 = (y1 >= 0) & (y1 < H)
            cx0 = jnp.clip(x0, 0, W - 1)
            cx1 = jnp.clip(x1, 0, W - 1)
            cy0 = jnp.clip(y0, 0, H - 1) * W
            cy1 = jnp.clip(y1, 0, H - 1) * W
            idx4[0, j, sl] = cy0 + cx0
            idx4[1, j, sl] = cy0 + cx1
            idx4[2, j, sl] = cy1 + cx0
            idx4[3, j, sl] = cy1 + cx1
            wm4[0, j, sl] = jnp.where(vx0 & vy0, nx * ny * m, zf)
            wm4[1, j, sl] = jnp.where(vx1 & vy0, ax * ny * m, zf)
            wm4[2, j, sl] = jnp.where(vx0 & vy1, nx * ay * m, zf)
            wm4[3, j, sl] = jnp.where(vx1 & vy1, ax * ay * m, zf)
        return 0

    lax.fori_loop(0, NCHUNK, _pre, 0)

    plsc.subcore_barrier()

    for k in range(4):
        def _dscat(jj, _, k=k):
            j = jj * 2
            d0 = pltpu.async_copy(wm4.at[k, j], den.at[idx4.at[k, j]],
                                  sem_a, add=True)
            d1 = pltpu.async_copy(wm4.at[k, j + 1], den.at[idx4.at[k, j + 1]],
                                  sem_a, add=True)
            d0.wait()
            d1.wait()
            return 0

        lax.fori_loop(0, NCHUNK // 2, _dscat, 0)

    plsc.subcore_barrier()

    pltpu.sync_copy(den.at[pl.ds(base, SRC)], dinv)
    one = jnp.full((LANES,), 1.0, jnp.float32)
    epsv = jnp.full((LANES,), EPS, jnp.float32)

    def _inv(i, _):
        sl = pl.ds(i * LANES, LANES)
        dinv[sl] = one / (dinv[sl] + epsv)
        return 0

    lax.fori_loop(0, SRC // LANES, _inv, 0)

    def _chan(c, _):
        pltpu.sync_copy(inp.at[b, c, pl.ds(base, SRC)], inbuf)
        for k in range(4):
            def _scat(jj, _, k=k):
                j = jj * 8
                ds = []
                for jo in range(8):
                    for q in range(VPC):
                        o = (j + jo) * CHUNK + q * LANES
                        stage[pl.ds(o, LANES)] = (
                            inbuf[pl.ds(o, LANES)]
                            * wm4[k, j + jo, pl.ds(q * LANES, LANES)])
                    ds.append(pltpu.async_copy(
                        stage.at[pl.ds((j + jo) * CHUNK, CHUNK)],
                        acc.at[idx4.at[k, j + jo]], sem_a, add=True))
                for d in ds:
                    d.wait()
                return 0

            lax.fori_loop(0, NCHUNK // 8, _scat, 0)
        plsc.subcore_barrier()
        pltpu.sync_copy(acc.at[pl.ds(base, SRC)], inbuf)

        def _nrm(i, _):
            sl = pl.ds(i * LANES, LANES)
            inbuf[sl] = inbuf[sl] * dinv[sl]
            return 0

        lax.fori_loop(0, SRC // LANES, _nrm, 0)
        pltpu.sync_copy(inbuf, out.at[b, c, pl.ds(base, SRC)])
        _fill_zero(inbuf)
        pltpu.sync_copy(inbuf, acc.at[pl.ds(base, SRC)])
        plsc.subcore_barrier()
        return 0

    lax.fori_loop(0, C, _chan, 0)


def _softsplat_sc(inp, flow, met):
    mesh = plsc.VectorSubcoreMesh(
        core_axis_name="c", subcore_axis_name="s",
        num_cores=2, num_subcores=16)
    f = pl.kernel(
        _sc_body,
        out_type=jax.ShapeDtypeStruct((B, C, HW), jnp.float32),
        mesh=mesh,
        scratch_types=[
            pltpu.VMEM((4, NCHUNK, CHUNK), jnp.int32),
            pltpu.VMEM((4, NCHUNK, CHUNK), jnp.float32),
            pltpu.VMEM((SRC,), jnp.float32),
            pltpu.VMEM((SRC,), jnp.float32),
            pltpu.VMEM((SRC,), jnp.float32),
            pltpu.VMEM_SHARED((HW,), jnp.float32),
            pltpu.VMEM_SHARED((HW,), jnp.float32),
            pltpu.SemaphoreType.DMA,
        ],
    )
    return f(inp, flow, met)


def kernel(tenInput, tenFlow, tenMetric):
    inp = tenInput.reshape(B, C, HW)
    flow = tenFlow.reshape(B, 2, HW)
    met = tenMetric.reshape(B, HW)
    out = _softsplat_sc(inp, flow, met)
    return out.reshape(B, C, H, W)


# speedup vs baseline: 2.1026x; 1.0398x over previous
import jax
import jax.numpy as jnp
from jax import lax
from jax.experimental import pallas as pl
from jax.experimental.pallas import tpu as pltpu
from jax.experimental.pallas import tpu_sc as plsc

B = 2
C = 96
H = 384
W = 384
HW = H * W
NT = 16
SRC = HW // NT
CHUNK = 128
NCHUNK = SRC // CHUNK
LANES = 16
VPC = CHUNK // LANES
EPS = 1e-7


def _sc_body(inp, flow, met, out, idx4, wm4, inbuf, stage, dinv, acc, den,
             sem_a):
    b = lax.axis_index("c")
    t = lax.axis_index("s")
    base = t * SRC

    zf = jnp.full((LANES,), 0.0, dtype=jnp.float32)

    def _fill_zero(buf):
        def _z(i, _):
            buf[pl.ds(i * LANES, LANES)] = zf
            return 0
        lax.fori_loop(0, SRC // LANES, _z, 0)

    _fill_zero(inbuf)
    pltpu.sync_copy(inbuf, acc.at[pl.ds(base, SRC)])
    pltpu.sync_copy(inbuf, den.at[pl.ds(base, SRC)])

    pltpu.sync_copy(flow.at[b, 0, pl.ds(base, SRC)], inbuf)
    pltpu.sync_copy(flow.at[b, 1, pl.ds(base, SRC)], stage)
    pltpu.sync_copy(met.at[b, pl.ds(base, SRC)], dinv)

    iota = lax.iota(jnp.int32, LANES)

    def _pre(j, _):
        for q in range(VPC):
            i = j * VPC + q
            p0 = base + i * LANES
            sl = pl.ds(q * LANES, LANES)
            vsl = pl.ds(i * LANES, LANES)
            fx = ((p0 % W) + iota).astype(jnp.float32) + inbuf[vsl]
            fy = (p0 // W).astype(jnp.float32) + stage[vsl]
            x0 = fx.astype(jnp.int32)
            x0f = x0.astype(jnp.float32)
            bx = x0f > fx
            x0 = jnp.where(bx, x0 - 1, x0)
            x0f = jnp.where(bx, x0f - 1.0, x0f)
            y0 = fy.astype(jnp.int32)
            y0f = y0.astype(jnp.float32)
            by = y0f > fy
            y0 = jnp.where(by, y0 - 1, y0)
            y0f = jnp.where(by, y0f - 1.0, y0f)
            ax = fx - x0f
            ay = fy - y0f
            nx = 1.0 - ax
            ny = 1.0 - ay
            m = jnp.exp(dinv[vsl])
            x1 = x0 + 1
            y1 = y0 + 1
            vx0 = (x0 >= 0) & (x0 < W)
            vx1 = (x1 >= 0) & (x1 < W)
            vy0 = (y0 >= 0) & (y0 < H)
            vy1 = (y1 >= 0) & (y1 < H)
            cx0 = jnp.clip(x0, 0, W - 1)
            cx1 = jnp.clip(x1, 0, W - 1)
            cy0 = jnp.clip(y0, 0, H - 1) * W
            cy1 = jnp.clip(y1, 0, H - 1) * W
            idx4[0, j, sl] = cy0 + cx0
            idx4[1, j, sl] = cy0 + cx1
            idx4[2, j, sl] = cy1 + cx0
            idx4[3, j, sl] = cy1 + cx1
            wm4[0, j, sl] = jnp.where(vx0 & vy0, nx * ny * m, zf)
            wm4[1, j, sl] = jnp.where(vx1 & vy0, ax * ny * m, zf)
            wm4[2, j, sl] = jnp.where(vx0 & vy1, nx * ay * m, zf)
            wm4[3, j, sl] = jnp.where(vx1 & vy1, ax * ay * m, zf)
        return 0

    lax.fori_loop(0, NCHUNK, _pre, 0)

    plsc.subcore_barrier()

    for k in range(4):
        def _dscat(jj, _, k=k):
            j = jj * 2
            d0 = pltpu.async_copy(wm4.at[k, j], den.at[idx4.at[k, j]],
                                  sem_a, add=True)
            d1 = pltpu.async_copy(wm4.at[k, j + 1], den.at[idx4.at[k, j + 1]],
                                  sem_a, add=True)
            d0.wait()
            d1.wait()
            return 0

        lax.fori_loop(0, NCHUNK // 2, _dscat, 0)

    plsc.subcore_barrier()

    pltpu.sync_copy(den.at[pl.ds(base, SRC)], dinv)
    one = jnp.full((LANES,), 1.0, jnp.float32)
    epsv = jnp.full((LANES,), EPS, jnp.float32)

    def _inv(i, _):
        sl = pl.ds(i * LANES, LANES)
        dinv[sl] = one / (dinv[sl] + epsv)
        return 0

    lax.fori_loop(0, SRC // LANES, _inv, 0)

    def _chan(c, _):
        pltpu.sync_copy(inp.at[b, c, pl.ds(base, SRC)], inbuf)
        for k in range(4):
            def _scat(jj, _, k=k):
                j = jj * 12
                ds = []
                for jo in range(12):
                    for q in range(VPC):
                        o = (j + jo) * CHUNK + q * LANES
                        stage[pl.ds(o, LANES)] = (
                            inbuf[pl.ds(o, LANES)]
                            * wm4[k, j + jo, pl.ds(q * LANES, LANES)])
                    ds.append(pltpu.async_copy(
                        stage.at[pl.ds((j + jo) * CHUNK, CHUNK)],
                        acc.at[idx4.at[k, j + jo]], sem_a, add=True))
                for d in ds:
                    d.wait()
                return 0

            lax.fori_loop(0, NCHUNK // 12, _scat, 0)
        plsc.subcore_barrier()
        pltpu.sync_copy(acc.at[pl.ds(base, SRC)], inbuf)

        def _nrm(i, _):
            sl = pl.ds(i * LANES, LANES)
            inbuf[sl] = inbuf[sl] * dinv[sl]
            return 0

        lax.fori_loop(0, SRC // LANES, _nrm, 0)
        pltpu.sync_copy(inbuf, out.at[b, c, pl.ds(base, SRC)])
        _fill_zero(inbuf)
        pltpu.sync_copy(inbuf, acc.at[pl.ds(base, SRC)])
        plsc.subcore_barrier()
        return 0

    lax.fori_loop(0, C, _chan, 0)


def _softsplat_sc(inp, flow, met):
    mesh = plsc.VectorSubcoreMesh(
        core_axis_name="c", subcore_axis_name="s",
        num_cores=2, num_subcores=16)
    f = pl.kernel(
        _sc_body,
        out_type=jax.ShapeDtypeStruct((B, C, HW), jnp.float32),
        mesh=mesh,
        scratch_types=[
            pltpu.VMEM((4, NCHUNK, CHUNK), jnp.int32),
            pltpu.VMEM((4, NCHUNK, CHUNK), jnp.float32),
            pltpu.VMEM((SRC,), jnp.float32),
            pltpu.VMEM((SRC,), jnp.float32),
            pltpu.VMEM((SRC,), jnp.float32),
            pltpu.VMEM_SHARED((HW,), jnp.float32),
            pltpu.VMEM_SHARED((HW,), jnp.float32),
            pltpu.SemaphoreType.DMA,
        ],
    )
    return f(inp, flow, met)


def kernel(tenInput, tenFlow, tenMetric):
    inp = tenInput.reshape(B, C, HW)
    flow = tenFlow.reshape(B, 2, HW)
    met = tenMetric.reshape(B, HW)
    out = _softsplat_sc(inp, flow, met)
    return out.reshape(B, C, H, W)


# speedup vs baseline: 2.1885x; 1.0408x over previous
import jax
import jax.numpy as jnp
from jax import lax
from jax.experimental import pallas as pl
from jax.experimental.pallas import tpu as pltpu
from jax.experimental.pallas import tpu_sc as plsc

B = 2
C = 96
H = 384
W = 384
HW = H * W
NT = 16
SRC = HW // NT
CHUNK = 128
NCHUNK = SRC // CHUNK
LANES = 16
VPC = CHUNK // LANES
EPS = 1e-7


def _sc_body(inp, flow, met, out, idx4, wm4, inbuf, stage, dinv, acc, den,
             sem_a):
    b = lax.axis_index("c")
    t = lax.axis_index("s")
    base = t * SRC

    zf = jnp.full((LANES,), 0.0, dtype=jnp.float32)

    def _fill_zero(buf):
        def _z(i, _):
            buf[pl.ds(i * LANES, LANES)] = zf
            return 0
        lax.fori_loop(0, SRC // LANES, _z, 0)

    _fill_zero(inbuf)
    pltpu.sync_copy(inbuf, acc.at[pl.ds(base, SRC)])
    pltpu.sync_copy(inbuf, den.at[pl.ds(base, SRC)])

    pltpu.sync_copy(flow.at[b, 0, pl.ds(base, SRC)], inbuf)
    pltpu.sync_copy(flow.at[b, 1, pl.ds(base, SRC)], stage)
    pltpu.sync_copy(met.at[b, pl.ds(base, SRC)], dinv)

    iota = lax.iota(jnp.int32, LANES)

    def _pre(j, _):
        for q in range(VPC):
            i = j * VPC + q
            p0 = base + i * LANES
            sl = pl.ds(q * LANES, LANES)
            vsl = pl.ds(i * LANES, LANES)
            fx = ((p0 % W) + iota).astype(jnp.float32) + inbuf[vsl]
            fy = (p0 // W).astype(jnp.float32) + stage[vsl]
            x0 = fx.astype(jnp.int32)
            x0f = x0.astype(jnp.float32)
            bx = x0f > fx
            x0 = jnp.where(bx, x0 - 1, x0)
            x0f = jnp.where(bx, x0f - 1.0, x0f)
            y0 = fy.astype(jnp.int32)
            y0f = y0.astype(jnp.float32)
            by = y0f > fy
            y0 = jnp.where(by, y0 - 1, y0)
            y0f = jnp.where(by, y0f - 1.0, y0f)
            ax = fx - x0f
            ay = fy - y0f
            nx = 1.0 - ax
            ny = 1.0 - ay
            m = jnp.exp(dinv[vsl])
            x1 = x0 + 1
            y1 = y0 + 1
            vx0 = (x0 >= 0) & (x0 < W)
            vx1 = (x1 >= 0) & (x1 < W)
            vy0 = (y0 >= 0) & (y0 < H)
            vy1 = (y1 >= 0) & (y1 < H)
            cx0 = jnp.clip(x0, 0, W - 1)
            cx1 = jnp.clip(x1, 0, W - 1)
            cy0 = jnp.clip(y0, 0, H - 1) * W
            cy1 = jnp.clip(y1, 0, H - 1) * W
            idx4[0, j, sl] = cy0 + cx0
            idx4[1, j, sl] = cy0 + cx1
            idx4[2, j, sl] = cy1 + cx0
            idx4[3, j, sl] = cy1 + cx1
            wm4[0, j, sl] = jnp.where(vx0 & vy0, nx * ny * m, zf)
            wm4[1, j, sl] = jnp.where(vx1 & vy0, ax * ny * m, zf)
            wm4[2, j, sl] = jnp.where(vx0 & vy1, nx * ay * m, zf)
            wm4[3, j, sl] = jnp.where(vx1 & vy1, ax * ay * m, zf)
        return 0

    lax.fori_loop(0, NCHUNK, _pre, 0)

    plsc.subcore_barrier()

    for k in range(4):
        def _dscat(jj, _, k=k):
            j = jj * 2
            d0 = pltpu.async_copy(wm4.at[k, j], den.at[idx4.at[k, j]],
                                  sem_a, add=True)
            d1 = pltpu.async_copy(wm4.at[k, j + 1], den.at[idx4.at[k, j + 1]],
                                  sem_a, add=True)
            d0.wait()
            d1.wait()
            return 0

        lax.fori_loop(0, NCHUNK // 2, _dscat, 0)

    plsc.subcore_barrier()

    pltpu.sync_copy(den.at[pl.ds(base, SRC)], dinv)
    one = jnp.full((LANES,), 1.0, jnp.float32)
    epsv = jnp.full((LANES,), EPS, jnp.float32)

    def _inv(i, _):
        sl = pl.ds(i * LANES, LANES)
        dinv[sl] = one / (dinv[sl] + epsv)
        return 0

    lax.fori_loop(0, SRC // LANES, _inv, 0)

    def _chan(c, _):
        pltpu.sync_copy(inp.at[b, c, pl.ds(base, SRC)], inbuf)
        for k in range(4):
            def _scat(jj, _, k=k):
                j = jj * 24
                ds = []
                for jo in range(24):
                    for q in range(VPC):
                        o = (j + jo) * CHUNK + q * LANES
                        stage[pl.ds(o, LANES)] = (
                            inbuf[pl.ds(o, LANES)]
                            * wm4[k, j + jo, pl.ds(q * LANES, LANES)])
                    ds.append(pltpu.async_copy(
                        stage.at[pl.ds((j + jo) * CHUNK, CHUNK)],
                        acc.at[idx4.at[k, j + jo]], sem_a, add=True))
                for d in ds:
                    d.wait()
                return 0

            lax.fori_loop(0, NCHUNK // 24, _scat, 0)
        plsc.subcore_barrier()
        pltpu.sync_copy(acc.at[pl.ds(base, SRC)], inbuf)

        def _nrm(i, _):
            sl = pl.ds(i * LANES, LANES)
            inbuf[sl] = inbuf[sl] * dinv[sl]
            return 0

        lax.fori_loop(0, SRC // LANES, _nrm, 0)
        pltpu.sync_copy(inbuf, out.at[b, c, pl.ds(base, SRC)])
        _fill_zero(inbuf)
        pltpu.sync_copy(inbuf, acc.at[pl.ds(base, SRC)])
        plsc.subcore_barrier()
        return 0

    lax.fori_loop(0, C, _chan, 0)


def _softsplat_sc(inp, flow, met):
    mesh = plsc.VectorSubcoreMesh(
        core_axis_name="c", subcore_axis_name="s",
        num_cores=2, num_subcores=16)
    f = pl.kernel(
        _sc_body,
        out_type=jax.ShapeDtypeStruct((B, C, HW), jnp.float32),
        mesh=mesh,
        scratch_types=[
            pltpu.VMEM((4, NCHUNK, CHUNK), jnp.int32),
            pltpu.VMEM((4, NCHUNK, CHUNK), jnp.float32),
            pltpu.VMEM((SRC,), jnp.float32),
            pltpu.VMEM((SRC,), jnp.float32),
            pltpu.VMEM((SRC,), jnp.float32),
            pltpu.VMEM_SHARED((HW,), jnp.float32),
            pltpu.VMEM_SHARED((HW,), jnp.float32),
            pltpu.SemaphoreType.DMA,
        ],
    )
    return f(inp, flow, met)


def kernel(tenInput, tenFlow, tenMetric):
    inp = tenInput.reshape(B, C, HW)
    flow = tenFlow.reshape(B, 2, HW)
    met = tenMetric.reshape(B, HW)
    out = _softsplat_sc(inp, flow, met)
    return out.reshape(B, C, H, W)


# speedup vs baseline: 2.1897x; 1.0006x over previous
import jax
import jax.numpy as jnp
from jax import lax
from jax.experimental import pallas as pl
from jax.experimental.pallas import tpu as pltpu
from jax.experimental.pallas import tpu_sc as plsc

B = 2
C = 96
H = 384
W = 384
HW = H * W
NT = 16
SRC = HW // NT
CHUNK = 128
NCHUNK = SRC // CHUNK
LANES = 16
VPC = CHUNK // LANES
EPS = 1e-7


def _sc_body(inp, flow, met, out, idx4, wm4, inbuf, stage, dinv, acc, den,
             sem_a):
    b = lax.axis_index("c")
    t = lax.axis_index("s")
    base = t * SRC

    zf = jnp.full((LANES,), 0.0, dtype=jnp.float32)

    def _fill_zero(buf):
        def _z(i, _):
            buf[pl.ds(i * LANES, LANES)] = zf
            return 0
        lax.fori_loop(0, SRC // LANES, _z, 0)

    _fill_zero(inbuf)
    pltpu.sync_copy(inbuf, acc.at[pl.ds(base, SRC)])
    pltpu.sync_copy(inbuf, den.at[pl.ds(base, SRC)])

    pltpu.sync_copy(flow.at[b, 0, pl.ds(base, SRC)], inbuf)
    pltpu.sync_copy(flow.at[b, 1, pl.ds(base, SRC)], stage)
    pltpu.sync_copy(met.at[b, pl.ds(base, SRC)], dinv)

    iota = lax.iota(jnp.int32, LANES)

    def _pre(j, _):
        for q in range(VPC):
            i = j * VPC + q
            p0 = base + i * LANES
            sl = pl.ds(q * LANES, LANES)
            vsl = pl.ds(i * LANES, LANES)
            fx = ((p0 % W) + iota).astype(jnp.float32) + inbuf[vsl]
            fy = (p0 // W).astype(jnp.float32) + stage[vsl]
            x0 = fx.astype(jnp.int32)
            x0f = x0.astype(jnp.float32)
            bx = x0f > fx
            x0 = jnp.where(bx, x0 - 1, x0)
            x0f = jnp.where(bx, x0f - 1.0, x0f)
            y0 = fy.astype(jnp.int32)
            y0f = y0.astype(jnp.float32)
            by = y0f > fy
            y0 = jnp.where(by, y0 - 1, y0)
            y0f = jnp.where(by, y0f - 1.0, y0f)
            ax = fx - x0f
            ay = fy - y0f
            nx = 1.0 - ax
            ny = 1.0 - ay
            m = jnp.exp(dinv[vsl])
            x1 = x0 + 1
            y1 = y0 + 1
            vx0 = (x0 >= 0) & (x0 < W)
            vx1 = (x1 >= 0) & (x1 < W)
            vy0 = (y0 >= 0) & (y0 < H)
            vy1 = (y1 >= 0) & (y1 < H)
            cx0 = jnp.clip(x0, 0, W - 1)
            cx1 = jnp.clip(x1, 0, W - 1)
            cy0 = jnp.clip(y0, 0, H - 1) * W
            cy1 = jnp.clip(y1, 0, H - 1) * W
            idx4[0, j, sl] = cy0 + cx0
            idx4[1, j, sl] = cy0 + cx1
            idx4[2, j, sl] = cy1 + cx0
            idx4[3, j, sl] = cy1 + cx1
            wm4[0, j, sl] = jnp.where(vx0 & vy0, nx * ny * m, zf)
            wm4[1, j, sl] = jnp.where(vx1 & vy0, ax * ny * m, zf)
            wm4[2, j, sl] = jnp.where(vx0 & vy1, nx * ay * m, zf)
            wm4[3, j, sl] = jnp.where(vx1 & vy1, ax * ay * m, zf)
        return 0

    lax.fori_loop(0, NCHUNK, _pre, 0)

    plsc.subcore_barrier()

    for k in range(4):
        def _dscat(jj, _, k=k):
            j = jj * 2
            d0 = pltpu.async_copy(wm4.at[k, j], den.at[idx4.at[k, j]],
                                  sem_a, add=True)
            d1 = pltpu.async_copy(wm4.at[k, j + 1], den.at[idx4.at[k, j + 1]],
                                  sem_a, add=True)
            d0.wait()
            d1.wait()
            return 0

        lax.fori_loop(0, NCHUNK // 2, _dscat, 0)

    plsc.subcore_barrier()

    pltpu.sync_copy(den.at[pl.ds(base, SRC)], dinv)
    one = jnp.full((LANES,), 1.0, jnp.float32)
    epsv = jnp.full((LANES,), EPS, jnp.float32)

    def _inv(i, _):
        sl = pl.ds(i * LANES, LANES)
        dinv[sl] = one / (dinv[sl] + epsv)
        return 0

    lax.fori_loop(0, SRC // LANES, _inv, 0)

    def _chan(c, _):
        pltpu.sync_copy(inp.at[b, c, pl.ds(base, SRC)], inbuf)
        for k in range(4):
            def _scat(jj, _, k=k):
                j = jj * 36
                ds = []
                for jo in range(36):
                    for q in range(VPC):
                        o = (j + jo) * CHUNK + q * LANES
                        stage[pl.ds(o, LANES)] = (
                            inbuf[pl.ds(o, LANES)]
                            * wm4[k, j + jo, pl.ds(q * LANES, LANES)])
                    ds.append(pltpu.async_copy(
                        stage.at[pl.ds((j + jo) * CHUNK, CHUNK)],
                        acc.at[idx4.at[k, j + jo]], sem_a, add=True))
                for d in ds:
                    d.wait()
                return 0

            lax.fori_loop(0, NCHUNK // 36, _scat, 0)
        plsc.subcore_barrier()
        pltpu.sync_copy(acc.at[pl.ds(base, SRC)], inbuf)

        def _nrm(i, _):
            sl = pl.ds(i * LANES, LANES)
            inbuf[sl] = inbuf[sl] * dinv[sl]
            return 0

        lax.fori_loop(0, SRC // LANES, _nrm, 0)
        pltpu.sync_copy(inbuf, out.at[b, c, pl.ds(base, SRC)])
        _fill_zero(inbuf)
        pltpu.sync_copy(inbuf, acc.at[pl.ds(base, SRC)])
        plsc.subcore_barrier()
        return 0

    lax.fori_loop(0, C, _chan, 0)


def _softsplat_sc(inp, flow, met):
    mesh = plsc.VectorSubcoreMesh(
        core_axis_name="c", subcore_axis_name="s",
        num_cores=2, num_subcores=16)
    f = pl.kernel(
        _sc_body,
        out_type=jax.ShapeDtypeStruct((B, C, HW), jnp.float32),
        mesh=mesh,
        scratch_types=[
            pltpu.VMEM((4, NCHUNK, CHUNK), jnp.int32),
            pltpu.VMEM((4, NCHUNK, CHUNK), jnp.float32),
            pltpu.VMEM((SRC,), jnp.float32),
            pltpu.VMEM((SRC,), jnp.float32),
            pltpu.VMEM((SRC,), jnp.float32),
            pltpu.VMEM_SHARED((HW,), jnp.float32),
            pltpu.VMEM_SHARED((HW,), jnp.float32),
            pltpu.SemaphoreType.DMA,
        ],
    )
    return f(inp, flow, met)


def kernel(tenInput, tenFlow, tenMetric):
    inp = tenInput.reshape(B, C, HW)
    flow = tenFlow.reshape(B, 2, HW)
    met = tenMetric.reshape(B, HW)
    out = _softsplat_sc(inp, flow, met)
    return out.reshape(B, C, H, W)
